# Initial kernel scaffold; baseline (speedup 1.0000x reference)
import jax
import jax.numpy as jnp
from jax.experimental import pallas as pl


def kernel(y_pred, y_true, id_mask):
    raise NotImplementedError("write your pallas kernel here")


# speedup vs baseline: 1.2563x; 1.2563x over previous
import jax
import jax.numpy as jnp
from jax import lax
from jax.experimental import pallas as pl
from jax.experimental.pallas import tpu as pltpu
from jax.experimental.pallas import tpu_sc as plsc

N = 32768
C = 128
G = 64

_ROWS = 512
_NT = 16
_E = N // _NT
_L = 16


def _loss_body(x_ref, yt_ref, out_ref):
    x = x_ref[...]
    yt = yt_ref[...]
    m = jnp.max(x, axis=1, keepdims=True)
    e = jnp.exp(x - m)
    s = jnp.sum(e, axis=1, keepdims=True)
    logz = jnp.log(s) + m
    cls = lax.broadcasted_iota(jnp.int32, x.shape, 1)
    t = jnp.sum(jnp.where(cls == yt, x, 0.0), axis=1, keepdims=True)
    out_ref[...] = logz - t


def _per_sample_loss(y_pred, y_true):
    grid = N // _ROWS
    return pl.pallas_call(
        _loss_body,
        grid=(grid,),
        in_specs=[
            pl.BlockSpec((_ROWS, C), lambda i: (i, 0)),
            pl.BlockSpec((_ROWS, 1), lambda i: (i, 0)),
        ],
        out_specs=pl.BlockSpec((_ROWS, 1), lambda i: (i, 0)),
        out_shape=jax.ShapeDtypeStruct((N, 1), jnp.float32),
    )(y_pred, y_true.reshape(N, 1))


def _segment_body(loss_hbm, ids_hbm, out_hbm,
                  loss_v, ids_v, sums_v, cnts_v, comb_v, tmp_v, out_v,
                  shared):
    s = lax.axis_index("s")
    base = s * _E
    pltpu.sync_copy(loss_hbm.at[pl.ds(base, _E)], loss_v)
    pltpu.sync_copy(ids_hbm.at[pl.ds(base, _E)], ids_v)

    zeros = jnp.zeros((_L,), jnp.float32)
    for r in range(_L):
        for j in range(G // _L):
            sums_v[pl.ds(r * G + j * _L, _L)] = zeros
            cnts_v[pl.ds(r * G + j * _L, _L)] = zeros

    lane_off = lax.iota(jnp.int32, _L) * G
    ones = jnp.ones((_L,), jnp.float32)

    def body(i, carry):
        l = loss_v[pl.ds(i * _L, _L)]
        g = ids_v[pl.ds(i * _L, _L)]
        flat = lane_off + g
        plsc.addupdate_scatter(sums_v, [flat], l)
        plsc.addupdate_scatter(cnts_v, [flat], ones)
        return carry

    lax.fori_loop(0, _E // _L, body, 0)

    for j in range(G // _L):
        acc_s = sums_v[pl.ds(j * _L, _L)]
        acc_c = cnts_v[pl.ds(j * _L, _L)]
        for r in range(1, _L):
            acc_s = acc_s + sums_v[pl.ds(r * G + j * _L, _L)]
            acc_c = acc_c + cnts_v[pl.ds(r * G + j * _L, _L)]
        comb_v[pl.ds(j * _L, _L)] = acc_s
        comb_v[pl.ds(G + j * _L, _L)] = acc_c

    pltpu.sync_copy(comb_v, shared.at[s])
    plsc.subcore_barrier()

    @pl.when(s == 0)
    def _():
        for t in range(1, _NT):
            pltpu.sync_copy(shared.at[t], tmp_v)
            for j in range(2 * G // _L):
                comb_v[pl.ds(j * _L, _L)] = (
                    comb_v[pl.ds(j * _L, _L)] + tmp_v[pl.ds(j * _L, _L)])
        total = jnp.zeros((_L,), jnp.float32)
        for j in range(G // _L):
            sv = comb_v[pl.ds(j * _L, _L)]
            cv = comb_v[pl.ds(G + j * _L, _L)]
            gl = jnp.where(cv > 0.0, sv / jnp.maximum(cv, 1.0), 0.0)
            total = total + gl
        tot = jnp.sum(total, axis=0)
        out_v[pl.ds(0, _L)] = jnp.full((_L,), tot, jnp.float32)
        pltpu.sync_copy(out_v, out_hbm)


def _segment_total(loss, ids):
    mesh = plsc.VectorSubcoreMesh(
        core_axis_name="c", subcore_axis_name="s", num_cores=1)
    kfn = pl.kernel(
        _segment_body,
        out_type=jax.ShapeDtypeStruct((_L,), jnp.float32),
        mesh=mesh,
        compiler_params=pltpu.CompilerParams(needs_layout_passes=False),
        scratch_types=[
            pltpu.VMEM((_E,), jnp.float32),
            pltpu.VMEM((_E,), jnp.int32),
            pltpu.VMEM((_L * G,), jnp.float32),
            pltpu.VMEM((_L * G,), jnp.float32),
            pltpu.VMEM((2 * G,), jnp.float32),
            pltpu.VMEM((2 * G,), jnp.float32),
            pltpu.VMEM((_L,), jnp.float32),
            pltpu.VMEM_SHARED((_NT, 2 * G), jnp.float32),
        ],
    )
    return kfn(loss, ids)


def kernel(y_pred, y_true, id_mask):
    loss = _per_sample_loss(y_pred, y_true.astype(jnp.int32))
    out16 = _segment_total(loss.reshape(N),
                           id_mask.reshape(N).astype(jnp.int32))
    return out16[0]


# speedup vs baseline: 1.7249x; 1.3730x over previous
import jax
import jax.numpy as jnp
from jax import lax
from jax.experimental import pallas as pl
from jax.experimental.pallas import tpu as pltpu
from jax.experimental.pallas import tpu_sc as plsc

N = 32768
C = 128
G = 64

_ROWS = 2048
_NT = 16
_E = N // _NT
_L = 16


def _loss_body(x_ref, yt_ref, out_ref):
    x = x_ref[...]
    yt = yt_ref[...]
    m = jnp.max(x, axis=1, keepdims=True)
    e = jnp.exp(x - m)
    s = jnp.sum(e, axis=1, keepdims=True)
    logz = jnp.log(s) + m
    cls = lax.broadcasted_iota(jnp.int32, x.shape, 1)
    t = jnp.sum(jnp.where(cls == yt, x, 0.0), axis=1, keepdims=True)
    out_ref[...] = logz - t


def _per_sample_loss(y_pred, y_true):
    grid = N // _ROWS
    return pl.pallas_call(
        _loss_body,
        grid=(grid,),
        in_specs=[
            pl.BlockSpec((_ROWS, C), lambda i: (i, 0)),
            pl.BlockSpec((_ROWS, 1), lambda i: (i, 0)),
        ],
        out_specs=pl.BlockSpec((_ROWS, 1), lambda i: (i, 0)),
        out_shape=jax.ShapeDtypeStruct((N, 1), jnp.float32),
    )(y_pred, y_true.reshape(N, 1))


def _segment_body(loss_hbm, ids_hbm, out_hbm,
                  loss_v, ids_v, sums_v, cnts_v, comb_v, tmp_v, out_v,
                  shared):
    s = lax.axis_index("s")
    base = s * _E
    pltpu.sync_copy(loss_hbm.at[pl.ds(base, _E)], loss_v)
    pltpu.sync_copy(ids_hbm.at[pl.ds(base, _E)], ids_v)

    zeros = jnp.zeros((_L,), jnp.float32)
    for r in range(_L):
        for j in range(G // _L):
            sums_v[pl.ds(r * G + j * _L, _L)] = zeros
            cnts_v[pl.ds(r * G + j * _L, _L)] = zeros

    lane_off = lax.iota(jnp.int32, _L) * G
    ones = jnp.ones((_L,), jnp.float32)

    def body(i, carry):
        l = loss_v[pl.ds(i * _L, _L)]
        g = ids_v[pl.ds(i * _L, _L)]
        flat = lane_off + g
        plsc.addupdate_scatter(sums_v, [flat], l)
        plsc.addupdate_scatter(cnts_v, [flat], ones)
        return carry

    lax.fori_loop(0, _E // _L, body, 0)

    for j in range(G // _L):
        acc_s = sums_v[pl.ds(j * _L, _L)]
        acc_c = cnts_v[pl.ds(j * _L, _L)]
        for r in range(1, _L):
            acc_s = acc_s + sums_v[pl.ds(r * G + j * _L, _L)]
            acc_c = acc_c + cnts_v[pl.ds(r * G + j * _L, _L)]
        comb_v[pl.ds(j * _L, _L)] = acc_s
        comb_v[pl.ds(G + j * _L, _L)] = acc_c

    pltpu.sync_copy(comb_v, shared.at[s])
    plsc.subcore_barrier()

    @pl.when(s == 0)
    def _():
        for t in range(1, _NT):
            pltpu.sync_copy(shared.at[t], tmp_v)
            for j in range(2 * G // _L):
                comb_v[pl.ds(j * _L, _L)] = (
                    comb_v[pl.ds(j * _L, _L)] + tmp_v[pl.ds(j * _L, _L)])
        total = jnp.zeros((_L,), jnp.float32)
        for j in range(G // _L):
            sv = comb_v[pl.ds(j * _L, _L)]
            cv = comb_v[pl.ds(G + j * _L, _L)]
            gl = jnp.where(cv > 0.0, sv / jnp.maximum(cv, 1.0), 0.0)
            total = total + gl
        tot = jnp.sum(total, axis=0)
        out_v[pl.ds(0, _L)] = jnp.full((_L,), tot, jnp.float32)
        pltpu.sync_copy(out_v, out_hbm)


def _segment_total(loss, ids):
    mesh = plsc.VectorSubcoreMesh(
        core_axis_name="c", subcore_axis_name="s", num_cores=1)
    kfn = pl.kernel(
        _segment_body,
        out_type=jax.ShapeDtypeStruct((_L,), jnp.float32),
        mesh=mesh,
        compiler_params=pltpu.CompilerParams(needs_layout_passes=False),
        scratch_types=[
            pltpu.VMEM((_E,), jnp.float32),
            pltpu.VMEM((_E,), jnp.int32),
            pltpu.VMEM((_L * G,), jnp.float32),
            pltpu.VMEM((_L * G,), jnp.float32),
            pltpu.VMEM((2 * G,), jnp.float32),
            pltpu.VMEM((2 * G,), jnp.float32),
            pltpu.VMEM((_L,), jnp.float32),
            pltpu.VMEM_SHARED((_NT, 2 * G), jnp.float32),
        ],
    )
    return kfn(loss, ids)


def kernel(y_pred, y_true, id_mask):
    loss = _per_sample_loss(y_pred, y_true.astype(jnp.int32))
    out16 = _segment_total(loss.reshape(N),
                           id_mask.reshape(N).astype(jnp.int32))
    return out16[0]


# speedup vs baseline: 1.8328x; 1.0625x over previous
import jax
import jax.numpy as jnp
from jax import lax
from jax.experimental import pallas as pl
from jax.experimental.pallas import tpu as pltpu
from jax.experimental.pallas import tpu_sc as plsc

N = 32768
C = 128
G = 64

_ROWS = 4096
_NT = 16
_E = N // _NT
_L = 16


def _loss_body(x_ref, yt_ref, out_ref):
    x = x_ref[...]
    yt = yt_ref[...]
    m = jnp.max(x, axis=1, keepdims=True)
    e = jnp.exp(x - m)
    s = jnp.sum(e, axis=1, keepdims=True)
    logz = jnp.log(s) + m
    cls = lax.broadcasted_iota(jnp.int32, x.shape, 1)
    t = jnp.sum(jnp.where(cls == yt, x, 0.0), axis=1, keepdims=True)
    out_ref[...] = logz - t


def _per_sample_loss(y_pred, y_true):
    grid = N // _ROWS
    return pl.pallas_call(
        _loss_body,
        grid=(grid,),
        in_specs=[
            pl.BlockSpec((_ROWS, C), lambda i: (i, 0)),
            pl.BlockSpec((_ROWS, 1), lambda i: (i, 0)),
        ],
        out_specs=pl.BlockSpec((_ROWS, 1), lambda i: (i, 0)),
        out_shape=jax.ShapeDtypeStruct((N, 1), jnp.float32),
    )(y_pred, y_true.reshape(N, 1))


def _segment_body(loss_hbm, ids_hbm, out_hbm,
                  loss_v, ids_v, sums_v, cnts_v, comb_v, tmp_v, out_v,
                  shared):
    s = lax.axis_index("s")
    base = s * _E
    pltpu.sync_copy(loss_hbm.at[pl.ds(base, _E)], loss_v)
    pltpu.sync_copy(ids_hbm.at[pl.ds(base, _E)], ids_v)

    zeros = jnp.zeros((_L,), jnp.float32)
    for r in range(_L):
        for j in range(G // _L):
            sums_v[pl.ds(r * G + j * _L, _L)] = zeros
            cnts_v[pl.ds(r * G + j * _L, _L)] = zeros

    lane_off = lax.iota(jnp.int32, _L) * G
    ones = jnp.ones((_L,), jnp.float32)

    def body(i, carry):
        l = loss_v[pl.ds(i * _L, _L)]
        g = ids_v[pl.ds(i * _L, _L)]
        flat = lane_off + g
        plsc.addupdate_scatter(sums_v, [flat], l)
        plsc.addupdate_scatter(cnts_v, [flat], ones)
        return carry

    lax.fori_loop(0, _E // _L, body, 0)

    for j in range(G // _L):
        acc_s = sums_v[pl.ds(j * _L, _L)]
        acc_c = cnts_v[pl.ds(j * _L, _L)]
        for r in range(1, _L):
            acc_s = acc_s + sums_v[pl.ds(r * G + j * _L, _L)]
            acc_c = acc_c + cnts_v[pl.ds(r * G + j * _L, _L)]
        comb_v[pl.ds(j * _L, _L)] = acc_s
        comb_v[pl.ds(G + j * _L, _L)] = acc_c

    pltpu.sync_copy(comb_v, shared.at[s])
    plsc.subcore_barrier()

    @pl.when(s == 0)
    def _():
        for t in range(1, _NT):
            pltpu.sync_copy(shared.at[t], tmp_v)
            for j in range(2 * G // _L):
                comb_v[pl.ds(j * _L, _L)] = (
                    comb_v[pl.ds(j * _L, _L)] + tmp_v[pl.ds(j * _L, _L)])
        total = jnp.zeros((_L,), jnp.float32)
        for j in range(G // _L):
            sv = comb_v[pl.ds(j * _L, _L)]
            cv = comb_v[pl.ds(G + j * _L, _L)]
            gl = jnp.where(cv > 0.0, sv / jnp.maximum(cv, 1.0), 0.0)
            total = total + gl
        tot = jnp.sum(total, axis=0)
        out_v[pl.ds(0, _L)] = jnp.full((_L,), tot, jnp.float32)
        pltpu.sync_copy(out_v, out_hbm)


def _segment_total(loss, ids):
    mesh = plsc.VectorSubcoreMesh(
        core_axis_name="c", subcore_axis_name="s", num_cores=1)
    kfn = pl.kernel(
        _segment_body,
        out_type=jax.ShapeDtypeStruct((_L,), jnp.float32),
        mesh=mesh,
        compiler_params=pltpu.CompilerParams(needs_layout_passes=False),
        scratch_types=[
            pltpu.VMEM((_E,), jnp.float32),
            pltpu.VMEM((_E,), jnp.int32),
            pltpu.VMEM((_L * G,), jnp.float32),
            pltpu.VMEM((_L * G,), jnp.float32),
            pltpu.VMEM((2 * G,), jnp.float32),
            pltpu.VMEM((2 * G,), jnp.float32),
            pltpu.VMEM((_L,), jnp.float32),
            pltpu.VMEM_SHARED((_NT, 2 * G), jnp.float32),
        ],
    )
    return kfn(loss, ids)


def kernel(y_pred, y_true, id_mask):
    loss = _per_sample_loss(y_pred, y_true.astype(jnp.int32))
    out16 = _segment_total(loss.reshape(N),
                           id_mask.reshape(N).astype(jnp.int32))
    return out16[0]


# speedup vs baseline: 1.8471x; 1.0078x over previous
import jax
import jax.numpy as jnp
from jax import lax
from jax.experimental import pallas as pl
from jax.experimental.pallas import tpu as pltpu
from jax.experimental.pallas import tpu_sc as plsc

N = 32768
C = 128
G = 64

_ROWS = 8192
_NT = 16
_E = N // _NT
_L = 16


def _loss_body(x_ref, yt_ref, out_ref):
    x = x_ref[...]
    yt = yt_ref[...]
    m = jnp.max(x, axis=1, keepdims=True)
    e = jnp.exp(x - m)
    s = jnp.sum(e, axis=1, keepdims=True)
    logz = jnp.log(s) + m
    cls = lax.broadcasted_iota(jnp.int32, x.shape, 1)
    t = jnp.sum(jnp.where(cls == yt, x, 0.0), axis=1, keepdims=True)
    out_ref[...] = logz - t


def _per_sample_loss(y_pred, y_true):
    grid = N // _ROWS
    return pl.pallas_call(
        _loss_body,
        grid=(grid,),
        in_specs=[
            pl.BlockSpec((_ROWS, C), lambda i: (i, 0)),
            pl.BlockSpec((_ROWS, 1), lambda i: (i, 0)),
        ],
        out_specs=pl.BlockSpec((_ROWS, 1), lambda i: (i, 0)),
        out_shape=jax.ShapeDtypeStruct((N, 1), jnp.float32),
    )(y_pred, y_true.reshape(N, 1))


def _segment_body(loss_hbm, ids_hbm, out_hbm,
                  loss_v, ids_v, sums_v, cnts_v, comb_v, tmp_v, out_v,
                  shared):
    s = lax.axis_index("s")
    base = s * _E
    pltpu.sync_copy(loss_hbm.at[pl.ds(base, _E)], loss_v)
    pltpu.sync_copy(ids_hbm.at[pl.ds(base, _E)], ids_v)

    zeros = jnp.zeros((_L,), jnp.float32)
    for r in range(_L):
        for j in range(G // _L):
            sums_v[pl.ds(r * G + j * _L, _L)] = zeros
            cnts_v[pl.ds(r * G + j * _L, _L)] = zeros

    lane_off = lax.iota(jnp.int32, _L) * G
    ones = jnp.ones((_L,), jnp.float32)

    def body(i, carry):
        l = loss_v[pl.ds(i * _L, _L)]
        g = ids_v[pl.ds(i * _L, _L)]
        flat = lane_off + g
        plsc.addupdate_scatter(sums_v, [flat], l)
        plsc.addupdate_scatter(cnts_v, [flat], ones)
        return carry

    lax.fori_loop(0, _E // _L, body, 0)

    for j in range(G // _L):
        acc_s = sums_v[pl.ds(j * _L, _L)]
        acc_c = cnts_v[pl.ds(j * _L, _L)]
        for r in range(1, _L):
            acc_s = acc_s + sums_v[pl.ds(r * G + j * _L, _L)]
            acc_c = acc_c + cnts_v[pl.ds(r * G + j * _L, _L)]
        comb_v[pl.ds(j * _L, _L)] = acc_s
        comb_v[pl.ds(G + j * _L, _L)] = acc_c

    pltpu.sync_copy(comb_v, shared.at[s])
    plsc.subcore_barrier()

    @pl.when(s == 0)
    def _():
        for t in range(1, _NT):
            pltpu.sync_copy(shared.at[t], tmp_v)
            for j in range(2 * G // _L):
                comb_v[pl.ds(j * _L, _L)] = (
                    comb_v[pl.ds(j * _L, _L)] + tmp_v[pl.ds(j * _L, _L)])
        total = jnp.zeros((_L,), jnp.float32)
        for j in range(G // _L):
            sv = comb_v[pl.ds(j * _L, _L)]
            cv = comb_v[pl.ds(G + j * _L, _L)]
            gl = jnp.where(cv > 0.0, sv / jnp.maximum(cv, 1.0), 0.0)
            total = total + gl
        tot = jnp.sum(total, axis=0)
        out_v[pl.ds(0, _L)] = jnp.full((_L,), tot, jnp.float32)
        pltpu.sync_copy(out_v, out_hbm)


def _segment_total(loss, ids):
    mesh = plsc.VectorSubcoreMesh(
        core_axis_name="c", subcore_axis_name="s", num_cores=1)
    kfn = pl.kernel(
        _segment_body,
        out_type=jax.ShapeDtypeStruct((_L,), jnp.float32),
        mesh=mesh,
        compiler_params=pltpu.CompilerParams(needs_layout_passes=False),
        scratch_types=[
            pltpu.VMEM((_E,), jnp.float32),
            pltpu.VMEM((_E,), jnp.int32),
            pltpu.VMEM((_L * G,), jnp.float32),
            pltpu.VMEM((_L * G,), jnp.float32),
            pltpu.VMEM((2 * G,), jnp.float32),
            pltpu.VMEM((2 * G,), jnp.float32),
            pltpu.VMEM((_L,), jnp.float32),
            pltpu.VMEM_SHARED((_NT, 2 * G), jnp.float32),
        ],
    )
    return kfn(loss, ids)


def kernel(y_pred, y_true, id_mask):
    loss = _per_sample_loss(y_pred, y_true.astype(jnp.int32))
    out16 = _segment_total(loss.reshape(N),
                           id_mask.reshape(N).astype(jnp.int32))
    return out16[0]


# speedup vs baseline: 2.4307x; 1.3160x over previous
import jax
import jax.numpy as jnp
from jax import lax
from jax.experimental import pallas as pl
from jax.experimental.pallas import tpu as pltpu
from jax.experimental.pallas import tpu_sc as plsc

N = 32768
C = 128
G = 64

_ROWS = 8192
_NT = 16
_E = N // _NT
_L = 16


def _logz_body(x_ref, out_ref):
    x = x_ref[...]
    m = jnp.max(x, axis=1)
    e = jnp.exp(x - m[:, None])
    s = jnp.sum(e, axis=1)
    out_ref[...] = jnp.log(s) + m


def _logz(y_pred):
    grid = N // _ROWS
    return pl.pallas_call(
        _logz_body,
        grid=(grid,),
        in_specs=[pl.BlockSpec((_ROWS, C), lambda i: (i, 0))],
        out_specs=pl.BlockSpec((_ROWS,), lambda i: (i,)),
        out_shape=jax.ShapeDtypeStruct((N,), jnp.float32),
    )(y_pred)


def _segment_body(yp_hbm, yt_hbm, logz_hbm, ids_hbm, out_hbm,
                  yt_v, logz_v, ids_v, idx_v, tl_v,
                  sums_v, cnts_v, comb_v, tmp_v, out_v,
                  shared, sem):
    s = lax.axis_index("s")
    base = s * _E
    pltpu.sync_copy(yt_hbm.at[pl.ds(base, _E)], yt_v)
    pltpu.sync_copy(logz_hbm.at[pl.ds(base, _E)], logz_v)
    pltpu.sync_copy(ids_hbm.at[pl.ds(base, _E)], ids_v)

    lane = lax.iota(jnp.int32, _L)

    def ibody(i, carry):
        row = base + i * _L + lane
        idx_v[pl.ds(i * _L, _L)] = row * C + yt_v[pl.ds(i * _L, _L)]
        return carry

    lax.fori_loop(0, _E // _L, ibody, 0)

    pltpu.async_copy(yp_hbm.at[idx_v], tl_v, sem).wait()

    zeros = jnp.zeros((_L,), jnp.float32)
    for r in range(_L):
        for j in range(G // _L):
            sums_v[pl.ds(r * G + j * _L, _L)] = zeros
            cnts_v[pl.ds(r * G + j * _L, _L)] = zeros

    lane_off = lane * G
    ones = jnp.ones((_L,), jnp.float32)

    def body(i, carry):
        l = logz_v[pl.ds(i * _L, _L)] - tl_v[pl.ds(i * _L, _L)]
        g = ids_v[pl.ds(i * _L, _L)]
        flat = lane_off + g
        plsc.addupdate_scatter(sums_v, [flat], l)
        plsc.addupdate_scatter(cnts_v, [flat], ones)
        return carry

    lax.fori_loop(0, _E // _L, body, 0)

    for j in range(G // _L):
        acc_s = sums_v[pl.ds(j * _L, _L)]
        acc_c = cnts_v[pl.ds(j * _L, _L)]
        for r in range(1, _L):
            acc_s = acc_s + sums_v[pl.ds(r * G + j * _L, _L)]
            acc_c = acc_c + cnts_v[pl.ds(r * G + j * _L, _L)]
        comb_v[pl.ds(j * _L, _L)] = acc_s
        comb_v[pl.ds(G + j * _L, _L)] = acc_c

    pltpu.sync_copy(comb_v, shared.at[s])
    plsc.subcore_barrier()

    @pl.when(s == 0)
    def _():
        for t in range(1, _NT):
            pltpu.sync_copy(shared.at[t], tmp_v)
            for j in range(2 * G // _L):
                comb_v[pl.ds(j * _L, _L)] = (
                    comb_v[pl.ds(j * _L, _L)] + tmp_v[pl.ds(j * _L, _L)])
        total = jnp.zeros((_L,), jnp.float32)
        for j in range(G // _L):
            sv = comb_v[pl.ds(j * _L, _L)]
            cv = comb_v[pl.ds(G + j * _L, _L)]
            gl = jnp.where(cv > 0.0, sv / jnp.maximum(cv, 1.0), 0.0)
            total = total + gl
        tot = jnp.sum(total, axis=0)
        out_v[pl.ds(0, _L)] = jnp.full((_L,), tot, jnp.float32)
        pltpu.sync_copy(out_v, out_hbm)


def _segment_total(yp_flat, y_true, logz, ids):
    mesh = plsc.VectorSubcoreMesh(
        core_axis_name="c", subcore_axis_name="s", num_cores=1)
    kfn = pl.kernel(
        _segment_body,
        out_type=jax.ShapeDtypeStruct((_L,), jnp.float32),
        mesh=mesh,
        compiler_params=pltpu.CompilerParams(needs_layout_passes=False),
        scratch_types=[
            pltpu.VMEM((_E,), jnp.int32),
            pltpu.VMEM((_E,), jnp.float32),
            pltpu.VMEM((_E,), jnp.int32),
            pltpu.VMEM((_E,), jnp.int32),
            pltpu.VMEM((_E,), jnp.float32),
            pltpu.VMEM((_L * G,), jnp.float32),
            pltpu.VMEM((_L * G,), jnp.float32),
            pltpu.VMEM((2 * G,), jnp.float32),
            pltpu.VMEM((2 * G,), jnp.float32),
            pltpu.VMEM((_L,), jnp.float32),
            pltpu.VMEM_SHARED((_NT, 2 * G), jnp.float32),
            pltpu.SemaphoreType.DMA,
        ],
    )
    return kfn(yp_flat, y_true, logz, ids)


def kernel(y_pred, y_true, id_mask):
    logz = _logz(y_pred)
    out16 = _segment_total(y_pred.reshape(N * C),
                           y_true.reshape(N).astype(jnp.int32),
                           logz,
                           id_mask.reshape(N).astype(jnp.int32))
    return out16[0]


# speedup vs baseline: 2.5615x; 1.0538x over previous
import jax
import jax.numpy as jnp
from jax import lax
from jax.experimental import pallas as pl
from jax.experimental.pallas import tpu as pltpu
from jax.experimental.pallas import tpu_sc as plsc

N = 32768
C = 128
G = 64

_ROWS = 8192
_NT = 16
_E = N // _NT
_L = 16


def _logz_body(x_ref, out_ref):
    x = x_ref[...]
    m = jnp.max(x, axis=1)
    e = jnp.exp(x - m[:, None])
    s = jnp.sum(e, axis=1)
    out_ref[...] = jnp.log(s) + m


def _logz(y_pred):
    grid = N // _ROWS
    return pl.pallas_call(
        _logz_body,
        grid=(grid,),
        in_specs=[pl.BlockSpec((_ROWS, C), lambda i: (i, 0))],
        out_specs=pl.BlockSpec((_ROWS,), lambda i: (i,)),
        out_shape=jax.ShapeDtypeStruct((N,), jnp.float32),
    )(y_pred)


def _gather_body(yp_hbm, yt_hbm, tl_hbm, yt_v, idx_v, tl_v, sem):
    s = lax.axis_index("s")
    base = s * _E
    pltpu.sync_copy(yt_hbm.at[pl.ds(base, _E)], yt_v)

    lane = lax.iota(jnp.int32, _L)

    def ibody(i, carry):
        row = base + i * _L + lane
        idx_v[pl.ds(i * _L, _L)] = row * C + yt_v[pl.ds(i * _L, _L)]
        return carry

    lax.fori_loop(0, _E // _L, ibody, 0)

    pltpu.async_copy(yp_hbm.at[idx_v], tl_v, sem).wait()
    pltpu.sync_copy(tl_v, tl_hbm.at[pl.ds(base, _E)])


def _gather_true_logits(yp_flat, y_true):
    mesh = plsc.VectorSubcoreMesh(
        core_axis_name="c", subcore_axis_name="s", num_cores=1)
    kfn = pl.kernel(
        _gather_body,
        out_type=jax.ShapeDtypeStruct((N,), jnp.float32),
        mesh=mesh,
        compiler_params=pltpu.CompilerParams(needs_layout_passes=False),
        scratch_types=[
            pltpu.VMEM((_E,), jnp.int32),
            pltpu.VMEM((_E,), jnp.int32),
            pltpu.VMEM((_E,), jnp.float32),
            pltpu.SemaphoreType.DMA,
        ],
    )
    return kfn(yp_flat, y_true)


def _segment_body(tl_hbm, logz_hbm, ids_hbm, out_hbm,
                  tl_v, logz_v, ids_v,
                  sums_v, cnts_v, comb_v, tmp_v, out_v,
                  shared):
    s = lax.axis_index("s")
    base = s * _E
    pltpu.sync_copy(tl_hbm.at[pl.ds(base, _E)], tl_v)
    pltpu.sync_copy(logz_hbm.at[pl.ds(base, _E)], logz_v)
    pltpu.sync_copy(ids_hbm.at[pl.ds(base, _E)], ids_v)

    lane = lax.iota(jnp.int32, _L)

    zeros = jnp.zeros((_L,), jnp.float32)
    for r in range(_L):
        for j in range(G // _L):
            sums_v[pl.ds(r * G + j * _L, _L)] = zeros
            cnts_v[pl.ds(r * G + j * _L, _L)] = zeros

    lane_off = lane * G
    del lane
    ones = jnp.ones((_L,), jnp.float32)

    def body(i, carry):
        l = logz_v[pl.ds(i * _L, _L)] - tl_v[pl.ds(i * _L, _L)]
        g = ids_v[pl.ds(i * _L, _L)]
        flat = lane_off + g
        plsc.addupdate_scatter(sums_v, [flat], l)
        plsc.addupdate_scatter(cnts_v, [flat], ones)
        return carry

    lax.fori_loop(0, _E // _L, body, 0)

    for j in range(G // _L):
        acc_s = sums_v[pl.ds(j * _L, _L)]
        acc_c = cnts_v[pl.ds(j * _L, _L)]
        for r in range(1, _L):
            acc_s = acc_s + sums_v[pl.ds(r * G + j * _L, _L)]
            acc_c = acc_c + cnts_v[pl.ds(r * G + j * _L, _L)]
        comb_v[pl.ds(j * _L, _L)] = acc_s
        comb_v[pl.ds(G + j * _L, _L)] = acc_c

    pltpu.sync_copy(comb_v, shared.at[s])
    plsc.subcore_barrier()

    @pl.when(s == 0)
    def _():
        for t in range(1, _NT):
            pltpu.sync_copy(shared.at[t], tmp_v)
            for j in range(2 * G // _L):
                comb_v[pl.ds(j * _L, _L)] = (
                    comb_v[pl.ds(j * _L, _L)] + tmp_v[pl.ds(j * _L, _L)])
        total = jnp.zeros((_L,), jnp.float32)
        for j in range(G // _L):
            sv = comb_v[pl.ds(j * _L, _L)]
            cv = comb_v[pl.ds(G + j * _L, _L)]
            gl = jnp.where(cv > 0.0, sv / jnp.maximum(cv, 1.0), 0.0)
            total = total + gl
        tot = jnp.sum(total, axis=0)
        out_v[pl.ds(0, _L)] = jnp.full((_L,), tot, jnp.float32)
        pltpu.sync_copy(out_v, out_hbm)


def _segment_total(tl, logz, ids):
    mesh = plsc.VectorSubcoreMesh(
        core_axis_name="c", subcore_axis_name="s", num_cores=1)
    kfn = pl.kernel(
        _segment_body,
        out_type=jax.ShapeDtypeStruct((_L,), jnp.float32),
        mesh=mesh,
        compiler_params=pltpu.CompilerParams(needs_layout_passes=False),
        scratch_types=[
            pltpu.VMEM((_E,), jnp.float32),
            pltpu.VMEM((_E,), jnp.float32),
            pltpu.VMEM((_E,), jnp.int32),
            pltpu.VMEM((_L * G,), jnp.float32),
            pltpu.VMEM((_L * G,), jnp.float32),
            pltpu.VMEM((2 * G,), jnp.float32),
            pltpu.VMEM((2 * G,), jnp.float32),
            pltpu.VMEM((_L,), jnp.float32),
            pltpu.VMEM_SHARED((_NT, 2 * G), jnp.float32),
        ],
    )
    return kfn(tl, logz, ids)


def kernel(y_pred, y_true, id_mask):
    tl = _gather_true_logits(y_pred.reshape(N * C),
                             y_true.reshape(N).astype(jnp.int32))
    logz = _logz(y_pred)
    out16 = _segment_total(tl, logz, id_mask.reshape(N).astype(jnp.int32))
    return out16[0]


# speedup vs baseline: 3.4367x; 1.3416x over previous
import jax
import jax.numpy as jnp
from jax import lax
from jax.experimental import pallas as pl
from jax.experimental.pallas import tpu as pltpu
from jax.experimental.pallas import tpu_sc as plsc

N = 32768
C = 128
G = 64

_ROWS = 8192
_NT = 16
_E = N // _NT
_L = 16


def _tc_body(x_ref, ids_ref, s1_ref, cnt_ref):
    i = pl.program_id(0)
    x = x_ref[...]
    m = jnp.max(x, axis=1, keepdims=True)
    e = jnp.exp(x - m)
    s = jnp.sum(e, axis=1, keepdims=True)
    logz = jnp.log(s) + m
    ids = ids_ref[...]
    gi = lax.broadcasted_iota(jnp.int32, (G, _ROWS), 0)
    oh_t = jnp.where(gi == ids[None, :], 1.0, 0.0)
    b = jnp.concatenate([logz, jnp.ones_like(logz)], axis=1)
    r = jnp.dot(oh_t, b, preferred_element_type=jnp.float32)

    @pl.when(i == 0)
    def _():
        s1_ref[...] = jnp.zeros((G,), jnp.float32)
        cnt_ref[...] = jnp.zeros((G,), jnp.float32)

    s1_ref[...] += r[:, 0]
    cnt_ref[...] += r[:, 1]


def _logz_group_sums(y_pred, ids):
    grid = N // _ROWS
    return pl.pallas_call(
        _tc_body,
        grid=(grid,),
        in_specs=[
            pl.BlockSpec((_ROWS, C), lambda i: (i, 0)),
            pl.BlockSpec((_ROWS,), lambda i: (i,)),
        ],
        out_specs=[
            pl.BlockSpec((G,), lambda i: (0,)),
            pl.BlockSpec((G,), lambda i: (0,)),
        ],
        out_shape=[
            jax.ShapeDtypeStruct((G,), jnp.float32),
            jax.ShapeDtypeStruct((G,), jnp.float32),
        ],
    )(y_pred, ids)


def _gather_body(yp_hbm, yt_hbm, tl_hbm, yt_v, idx_v, tl_v, sem):
    s = lax.axis_index("s")
    base = s * _E
    pltpu.sync_copy(yt_hbm.at[pl.ds(base, _E)], yt_v)

    lane = lax.iota(jnp.int32, _L)

    def ibody(i, carry):
        row = base + i * _L + lane
        idx_v[pl.ds(i * _L, _L)] = row * C + yt_v[pl.ds(i * _L, _L)]
        return carry

    lax.fori_loop(0, _E // _L, ibody, 0)

    pltpu.async_copy(yp_hbm.at[idx_v], tl_v, sem).wait()
    pltpu.sync_copy(tl_v, tl_hbm.at[pl.ds(base, _E)])


def _gather_true_logits(yp_flat, y_true):
    mesh = plsc.VectorSubcoreMesh(
        core_axis_name="c", subcore_axis_name="s", num_cores=1)
    kfn = pl.kernel(
        _gather_body,
        out_type=jax.ShapeDtypeStruct((N,), jnp.float32),
        mesh=mesh,
        compiler_params=pltpu.CompilerParams(needs_layout_passes=False),
        scratch_types=[
            pltpu.VMEM((_E,), jnp.int32),
            pltpu.VMEM((_E,), jnp.int32),
            pltpu.VMEM((_E,), jnp.float32),
            pltpu.SemaphoreType.DMA,
        ],
    )
    return kfn(yp_flat, y_true)


def _segment_body(tl_hbm, ids_hbm, s1_hbm, cnt_hbm, out_hbm,
                  tl_v, ids_v, sums_v, comb_v, big_v, s1_v, cnt_v, out_v,
                  shared):
    s = lax.axis_index("s")
    base = s * _E
    pltpu.sync_copy(tl_hbm.at[pl.ds(base, _E)], tl_v)
    pltpu.sync_copy(ids_hbm.at[pl.ds(base, _E)], ids_v)

    zeros = jnp.zeros((_L,), jnp.float32)
    for r in range(_L):
        for j in range(G // _L):
            sums_v[pl.ds(r * G + j * _L, _L)] = zeros

    lane_off = lax.iota(jnp.int32, _L) * G

    def body(i, carry):
        t = tl_v[pl.ds(i * _L, _L)]
        g = ids_v[pl.ds(i * _L, _L)]
        plsc.addupdate_scatter(sums_v, [lane_off + g], t)
        return carry

    lax.fori_loop(0, _E // _L, body, 0)

    for j in range(G // _L):
        acc = sums_v[pl.ds(j * _L, _L)]
        for r in range(1, _L):
            acc = acc + sums_v[pl.ds(r * G + j * _L, _L)]
        comb_v[pl.ds(j * _L, _L)] = acc

    pltpu.sync_copy(comb_v, shared.at[pl.ds(s * G, G)])
    plsc.subcore_barrier()

    @pl.when(s == 0)
    def _():
        pltpu.sync_copy(shared, big_v)
        pltpu.sync_copy(s1_hbm, s1_v)
        pltpu.sync_copy(cnt_hbm, cnt_v)
        total = jnp.zeros((_L,), jnp.float32)
        for j in range(G // _L):
            s2 = big_v[pl.ds(j * _L, _L)]
            for t in range(1, _NT):
                s2 = s2 + big_v[pl.ds(t * G + j * _L, _L)]
            s1 = s1_v[pl.ds(j * _L, _L)]
            cv = cnt_v[pl.ds(j * _L, _L)]
            gl = jnp.where(cv > 0.0, (s1 - s2) / jnp.maximum(cv, 1.0), 0.0)
            total = total + gl
        tot = jnp.sum(total, axis=0)
        out_v[pl.ds(0, _L)] = jnp.full((_L,), tot, jnp.float32)
        pltpu.sync_copy(out_v, out_hbm)


def _segment_total(tl, ids, s1, cnt):
    mesh = plsc.VectorSubcoreMesh(
        core_axis_name="c", subcore_axis_name="s", num_cores=1)
    kfn = pl.kernel(
        _segment_body,
        out_type=jax.ShapeDtypeStruct((_L,), jnp.float32),
        mesh=mesh,
        compiler_params=pltpu.CompilerParams(needs_layout_passes=False),
        scratch_types=[
            pltpu.VMEM((_E,), jnp.float32),
            pltpu.VMEM((_E,), jnp.int32),
            pltpu.VMEM((_L * G,), jnp.float32),
            pltpu.VMEM((G,), jnp.float32),
            pltpu.VMEM((_NT * G,), jnp.float32),
            pltpu.VMEM((G,), jnp.float32),
            pltpu.VMEM((G,), jnp.float32),
            pltpu.VMEM((_L,), jnp.float32),
            pltpu.VMEM_SHARED((_NT * G,), jnp.float32),
        ],
    )
    return kfn(tl, ids, s1, cnt)


def kernel(y_pred, y_true, id_mask):
    ids = id_mask.reshape(N).astype(jnp.int32)
    tl = _gather_true_logits(y_pred.reshape(N * C),
                             y_true.reshape(N).astype(jnp.int32))
    s1, cnt = _logz_group_sums(y_pred, ids)
    out16 = _segment_total(tl, ids, s1, cnt)
    return out16[0]
